# baseline (speedup 1.0000x reference)
import functools

import jax
import jax.numpy as jnp
from jax import lax
from jax.experimental import pallas as pl
from jax.experimental.pallas import tpu as pltpu
from jax.experimental.pallas import tpu_sc as plsc

N = 10000
D = 128
E = 320000

NC = 2
NS = 16
L = 16
NW = NC * NS

K = 128
CHUNKS = 80
EPW = CHUNKS * K
E_PAD = NW * EPW
PNL = 16
NPANEL = CHUNKS // PNL

NR = 10240
CE = 1024
CS = 2048


def _sc_mesh():
    return plsc.VectorSubcoreMesh(
        core_axis_name="c", subcore_axis_name="s",
        num_cores=NC, num_subcores=NS)


def _deg_body(rf_ref, dm_ref):
    @pl.when(pl.program_id(0) == 0)
    def _():
        dm_ref[...] = jnp.zeros_like(dm_ref)

    rf = rf_ref[0]
    a = jnp.floor(rf * (1.0 / 128.0))
    b = rf - 128.0 * a
    iota = lax.broadcasted_iota(jnp.int32, (CE, 128), 1).astype(jnp.float32)
    oa = (a == iota).astype(jnp.float32)
    ob = (b == iota).astype(jnp.float32)
    dm_ref[...] += lax.dot_general(
        oa, ob, (((0,), (0,)), ((), ())),
        preferred_element_type=jnp.float32)


def _deg_mat(rows_f):
    nch = E_PAD // CE
    return pl.pallas_call(
        _deg_body,
        grid=(nch,),
        in_specs=[pl.BlockSpec((1, CE, 1), lambda i: (i, 0, 0))],
        out_specs=pl.BlockSpec((128, 128), lambda i: (0, 0)),
        out_shape=jax.ShapeDtypeStruct((128, 128), jnp.float32),
    )(rows_f)


def _dis(deg):
    return jnp.where(deg > 0, lax.rsqrt(jnp.maximum(deg, 1.0)), 0.0)


def _scale_matmul_body(deg_ref, x_ref, w_ref, z_ref):
    dis = _dis(deg_ref[...])
    y = jnp.dot(x_ref[...], w_ref[...], preferred_element_type=jnp.float32)
    z_ref[...] = dis * y


def _scale_matmul(deg, x_pad, W):
    blk = 1024
    return pl.pallas_call(
        _scale_matmul_body,
        grid=(NR // blk,),
        in_specs=[pl.BlockSpec((blk, 1), lambda i: (i, 0)),
                  pl.BlockSpec((blk, D), lambda i: (i, 0)),
                  pl.BlockSpec((D, D), lambda i: (0, 0))],
        out_specs=pl.BlockSpec((blk, D), lambda i: (i, 0)),
        out_shape=jax.ShapeDtypeStruct((NR, D), jnp.float32),
    )(deg, x_pad, W)


def _make_sc_gather():
    @functools.partial(
        pl.kernel,
        out_type=jax.ShapeDtypeStruct((E_PAD, D), jnp.float32),
        mesh=_sc_mesh(),
        scratch_types=[
            pltpu.VMEM((PNL, K), jnp.int32),
            pltpu.VMEM((PNL, K), jnp.int32),
            pltpu.VMEM((K, D), jnp.float32),
            pltpu.VMEM((K, D), jnp.float32),
            pltpu.SemaphoreType.DMA,
            pltpu.SemaphoreType.DMA,
        ],
    )
    def gather(col_hbm, z_hbm, g_hbm, cpa, cpb, buf0, buf1, sem0, sem1):
        c = lax.axis_index("c")
        s = lax.axis_index("s")
        w = s * NC + c
        base = w * EPW

        pltpu.sync_copy(col_hbm.at[w, pl.ds(0, PNL)], cpa)
        pltpu.async_copy(z_hbm.at[cpa.at[0]], buf0, sem0)

        pans = [(cpa,), (cpb,)]
        for p in range(NPANEL):
            (cA,) = pans[p % 2]
            (cB,) = pans[(p + 1) % 2]
            pbase = base + p * PNL * K
            if p < NPANEL - 1:
                pltpu.sync_copy(col_hbm.at[w, pl.ds((p + 1) * PNL, PNL)], cB)

            @pl.loop(0, PNL // 2 - 1, unroll=False)
            def pair(i, cA=cA, pbase=pbase):
                j0 = 2 * i
                pltpu.async_copy(z_hbm.at[cA.at[j0 + 1]], buf1, sem1)
                pltpu.make_async_copy(z_hbm.at[cA.at[j0]], buf0, sem0).wait()
                pltpu.sync_copy(buf0, g_hbm.at[pl.ds(pbase + j0 * K, K)])
                pltpu.async_copy(z_hbm.at[cA.at[j0 + 2]], buf0, sem0)
                pltpu.make_async_copy(z_hbm.at[cA.at[j0 + 1]], buf1, sem1).wait()
                pltpu.sync_copy(buf1, g_hbm.at[pl.ds(pbase + (j0 + 1) * K, K)])

            pltpu.async_copy(z_hbm.at[cA.at[PNL - 1]], buf1, sem1)
            pltpu.make_async_copy(z_hbm.at[cA.at[PNL - 2]], buf0, sem0).wait()
            pltpu.sync_copy(buf0, g_hbm.at[pl.ds(pbase + (PNL - 2) * K, K)])
            if p < NPANEL - 1:
                pltpu.async_copy(z_hbm.at[cB.at[0]], buf0, sem0)
            pltpu.make_async_copy(z_hbm.at[cA.at[PNL - 1]], buf1, sem1).wait()
            pltpu.sync_copy(buf1, g_hbm.at[pl.ds(pbase + (PNL - 1) * K, K)])

    return gather


_sc_gather_kernel = _make_sc_gather()


def _scatter_body(rows_ref, g_ref, acc_ref):
    @pl.when(pl.program_id(0) == 0)
    def _():
        acc_ref[...] = jnp.zeros_like(acc_ref)

    def body(j, _):
        r = rows_ref[0, 0, j]
        acc_ref[pl.ds(r, 1), :] += g_ref[pl.ds(j, 1), :]
        return 0
    lax.fori_loop(0, CS, body, 0)


def _tc_scatter(rows2, G):
    nch = E_PAD // CS
    return pl.pallas_call(
        _scatter_body,
        grid=(nch,),
        in_specs=[pl.BlockSpec((1, 1, CS), lambda i: (i, 0, 0),
                               memory_space=pltpu.SMEM),
                  pl.BlockSpec((CS, D), lambda i: (i, 0))],
        out_specs=pl.BlockSpec((NR, D), lambda i: (0, 0)),
        out_shape=jax.ShapeDtypeStruct((NR, D), jnp.float32),
    )(rows2, G)


def _epilogue_body(deg_ref, acc_ref, b_ref, o_ref):
    dis = _dis(deg_ref[...])
    v = dis * acc_ref[...] + b_ref[...]
    o_ref[...] = jnp.where(v >= 0, v, 0.2 * v)


def _epilogue(deg, acc, b2):
    blk = 1000
    return pl.pallas_call(
        _epilogue_body,
        grid=(N // blk,),
        in_specs=[pl.BlockSpec((blk, 1), lambda i: (i, 0)),
                  pl.BlockSpec((blk, D), lambda i: (i, 0)),
                  pl.BlockSpec((1, D), lambda i: (0, 0))],
        out_specs=pl.BlockSpec((blk, D), lambda i: (i, 0)),
        out_shape=jax.ShapeDtypeStruct((N, D), jnp.float32),
    )(deg, acc, b2)


def kernel(x, edge_index, W, b):
    ei = edge_index.astype(jnp.int32)
    pad = E_PAD - E
    row_p = jnp.concatenate([ei[0], jnp.full((pad,), N, jnp.int32)])
    col_p = jnp.concatenate([ei[1], jnp.full((pad,), N, jnp.int32)])

    rows_f = row_p.astype(jnp.float32).reshape(E_PAD // CE, CE, 1)
    deg_mat = _deg_mat(rows_f)
    deg = deg_mat.reshape(128 * 128, 1)[:NR]

    x_pad = jnp.concatenate(
        [x, jnp.zeros((NR - N, D), jnp.float32)], axis=0)
    z = _scale_matmul(deg, x_pad, W)

    col3 = col_p.reshape(NW, CHUNKS, K)
    G = _sc_gather_kernel(col3, z)

    rows2 = row_p.reshape(E_PAD // CS, 1, CS)
    acc = _tc_scatter(rows2, G)

    return _epilogue(deg[:N], acc[:N], b.reshape(1, D))


# speedup vs baseline: 1.8525x; 1.8525x over previous
import functools

import jax
import jax.numpy as jnp
from jax import lax
from jax.experimental import pallas as pl
from jax.experimental.pallas import tpu as pltpu
from jax.experimental.pallas import tpu_sc as plsc

N = 10000
D = 128
E = 320000

NC = 2
NS = 16
L = 16
NW = NC * NS

K = 128
CHUNKS = 80
EPW = CHUNKS * K
E_PAD = NW * EPW
PNL = 16
NPANEL = CHUNKS // PNL

NR = 10240
CE = 1024
CS = 2048


def _sc_mesh():
    return plsc.VectorSubcoreMesh(
        core_axis_name="c", subcore_axis_name="s",
        num_cores=NC, num_subcores=NS)


def _deg_body(rf_ref, dm_ref):
    @pl.when(pl.program_id(0) == 0)
    def _():
        dm_ref[...] = jnp.zeros_like(dm_ref)

    rf = rf_ref[0]
    a = jnp.floor(rf * (1.0 / 128.0))
    b = rf - 128.0 * a
    iota = lax.broadcasted_iota(jnp.int32, (CE, 128), 1).astype(jnp.float32)
    oa = (a == iota).astype(jnp.float32)
    ob = (b == iota).astype(jnp.float32)
    dm_ref[...] += lax.dot_general(
        oa, ob, (((0,), (0,)), ((), ())),
        preferred_element_type=jnp.float32)


def _deg_mat(rows_f):
    nch = E_PAD // CE
    return pl.pallas_call(
        _deg_body,
        grid=(nch,),
        in_specs=[pl.BlockSpec((1, CE, 1), lambda i: (i, 0, 0))],
        out_specs=pl.BlockSpec((128, 128), lambda i: (0, 0)),
        out_shape=jax.ShapeDtypeStruct((128, 128), jnp.float32),
    )(rows_f)


def _dis(deg):
    return jnp.where(deg > 0, lax.rsqrt(jnp.maximum(deg, 1.0)), 0.0)


def _scale_matmul_body(deg_ref, x_ref, w_ref, z_ref):
    dis = _dis(deg_ref[...])
    y = jnp.dot(x_ref[...], w_ref[...], preferred_element_type=jnp.float32)
    z_ref[...] = dis * y


def _scale_matmul(deg, x_pad, W):
    blk = 1024
    return pl.pallas_call(
        _scale_matmul_body,
        grid=(NR // blk,),
        in_specs=[pl.BlockSpec((blk, 1), lambda i: (i, 0)),
                  pl.BlockSpec((blk, D), lambda i: (i, 0)),
                  pl.BlockSpec((D, D), lambda i: (0, 0))],
        out_specs=pl.BlockSpec((blk, D), lambda i: (i, 0)),
        out_shape=jax.ShapeDtypeStruct((NR, D), jnp.float32),
    )(deg, x_pad, W)


def _make_sc_gather():
    @functools.partial(
        pl.kernel,
        out_type=jax.ShapeDtypeStruct((E_PAD, D), jnp.float32),
        mesh=_sc_mesh(),
        scratch_types=[
            pltpu.VMEM((PNL, K), jnp.int32),
            pltpu.VMEM((PNL, K), jnp.int32),
            pltpu.VMEM((K, D), jnp.float32),
            pltpu.VMEM((K, D), jnp.float32),
            pltpu.SemaphoreType.DMA,
            pltpu.SemaphoreType.DMA,
        ],
    )
    def gather(col_hbm, z_hbm, g_hbm, cpa, cpb, buf0, buf1, sem0, sem1):
        c = lax.axis_index("c")
        s = lax.axis_index("s")
        w = s * NC + c
        base = w * EPW

        pltpu.sync_copy(col_hbm.at[w, pl.ds(0, PNL)], cpa)
        pltpu.async_copy(z_hbm.at[cpa.at[0]], buf0, sem0)

        pans = [(cpa,), (cpb,)]
        for p in range(NPANEL):
            (cA,) = pans[p % 2]
            (cB,) = pans[(p + 1) % 2]
            pbase = base + p * PNL * K
            if p < NPANEL - 1:
                pltpu.sync_copy(col_hbm.at[w, pl.ds((p + 1) * PNL, PNL)], cB)

            @pl.loop(0, PNL // 2 - 1, unroll=False)
            def pair(i, cA=cA, pbase=pbase):
                j0 = 2 * i
                pltpu.async_copy(z_hbm.at[cA.at[j0 + 1]], buf1, sem1)
                pltpu.make_async_copy(z_hbm.at[cA.at[j0]], buf0, sem0).wait()
                pltpu.sync_copy(buf0, g_hbm.at[pl.ds(pbase + j0 * K, K)])
                pltpu.async_copy(z_hbm.at[cA.at[j0 + 2]], buf0, sem0)
                pltpu.make_async_copy(z_hbm.at[cA.at[j0 + 1]], buf1, sem1).wait()
                pltpu.sync_copy(buf1, g_hbm.at[pl.ds(pbase + (j0 + 1) * K, K)])

            pltpu.async_copy(z_hbm.at[cA.at[PNL - 1]], buf1, sem1)
            pltpu.make_async_copy(z_hbm.at[cA.at[PNL - 2]], buf0, sem0).wait()
            pltpu.sync_copy(buf0, g_hbm.at[pl.ds(pbase + (PNL - 2) * K, K)])
            if p < NPANEL - 1:
                pltpu.async_copy(z_hbm.at[cB.at[0]], buf0, sem0)
            pltpu.make_async_copy(z_hbm.at[cA.at[PNL - 1]], buf1, sem1).wait()
            pltpu.sync_copy(buf1, g_hbm.at[pl.ds(pbase + (PNL - 1) * K, K)])

    return gather


_sc_gather_kernel = _make_sc_gather()


NACC = 4


def _scatter_body(rows_ref, g_ref, *acc_refs):
    @pl.when(pl.program_id(0) == 0)
    def _():
        for a in acc_refs:
            a[...] = jnp.zeros_like(a)

    def body(j, _):
        j4 = NACC * j
        for q in range(NACC):
            r = rows_ref[0, 0, j4 + q]
            acc_refs[q][pl.ds(r, 1), :] += g_ref[pl.ds(j4 + q, 1), :]
        return 0
    lax.fori_loop(0, CS // NACC, body, 0)


def _tc_scatter(rows2, G):
    nch = E_PAD // CS
    return pl.pallas_call(
        _scatter_body,
        grid=(nch,),
        in_specs=[pl.BlockSpec((1, 1, CS), lambda i: (i, 0, 0),
                               memory_space=pltpu.SMEM),
                  pl.BlockSpec((CS, D), lambda i: (i, 0))],
        out_specs=[pl.BlockSpec((NR, D), lambda i: (0, 0))] * NACC,
        out_shape=[jax.ShapeDtypeStruct((NR, D), jnp.float32)] * NACC,
    )(rows2, G)


def _epilogue_body(deg_ref, b_ref, *refs):
    acc_refs = refs[:-1]
    o_ref = refs[-1]
    dis = _dis(deg_ref[...])
    acc = acc_refs[0][...]
    for a in acc_refs[1:]:
        acc = acc + a[...]
    v = dis * acc + b_ref[...]
    o_ref[...] = jnp.where(v >= 0, v, 0.2 * v)


def _epilogue(deg, accs, b2):
    blk = 1000
    fs = pl.BlockSpec((blk, D), lambda i: (i, 0))
    return pl.pallas_call(
        _epilogue_body,
        grid=(N // blk,),
        in_specs=[pl.BlockSpec((blk, 1), lambda i: (i, 0)),
                  pl.BlockSpec((1, D), lambda i: (0, 0))] + [fs] * NACC,
        out_specs=fs,
        out_shape=jax.ShapeDtypeStruct((N, D), jnp.float32),
    )(deg, b2, *accs)


def kernel(x, edge_index, W, b):
    ei = edge_index.astype(jnp.int32)
    pad = E_PAD - E
    row_p = jnp.concatenate([ei[0], jnp.full((pad,), N, jnp.int32)])
    col_p = jnp.concatenate([ei[1], jnp.full((pad,), N, jnp.int32)])

    rows_f = row_p.astype(jnp.float32).reshape(E_PAD // CE, CE, 1)
    deg_mat = _deg_mat(rows_f)
    deg = deg_mat.reshape(128 * 128, 1)[:NR]

    x_pad = jnp.concatenate(
        [x, jnp.zeros((NR - N, D), jnp.float32)], axis=0)
    z = _scale_matmul(deg, x_pad, W)

    col3 = col_p.reshape(NW, CHUNKS, K)
    G = _sc_gather_kernel(col3, z)

    rows2 = row_p.reshape(E_PAD // CS, 1, CS)
    accs = _tc_scatter(rows2, G)

    return _epilogue(deg[:N], [a[:N] for a in accs], b.reshape(1, D))


# speedup vs baseline: 2.1158x; 1.1422x over previous
import functools

import jax
import jax.numpy as jnp
from jax import lax
from jax.experimental import pallas as pl
from jax.experimental.pallas import tpu as pltpu
from jax.experimental.pallas import tpu_sc as plsc

N = 10000
D = 128
E = 320000

NC = 2
NS = 16
L = 16
NW = NC * NS

K = 128
CHUNKS = 80
EPW = CHUNKS * K
E_PAD = NW * EPW
PNL = 16
NPANEL = CHUNKS // PNL

NR = 10240
CE = 1024
CS = 2048


def _sc_mesh():
    return plsc.VectorSubcoreMesh(
        core_axis_name="c", subcore_axis_name="s",
        num_cores=NC, num_subcores=NS)


def _deg_body(rf_ref, dm_ref):
    @pl.when(pl.program_id(0) == 0)
    def _():
        dm_ref[...] = jnp.zeros_like(dm_ref)

    rf = rf_ref[0]
    a = jnp.floor(rf * (1.0 / 128.0))
    b = rf - 128.0 * a
    iota = lax.broadcasted_iota(jnp.int32, (CE, 128), 1).astype(jnp.float32)
    oa = (a == iota).astype(jnp.float32)
    ob = (b == iota).astype(jnp.float32)
    dm_ref[...] += lax.dot_general(
        oa, ob, (((0,), (0,)), ((), ())),
        preferred_element_type=jnp.float32)


def _deg_mat(rows_f):
    nch = E_PAD // CE
    return pl.pallas_call(
        _deg_body,
        grid=(nch,),
        in_specs=[pl.BlockSpec((1, CE, 1), lambda i: (i, 0, 0))],
        out_specs=pl.BlockSpec((128, 128), lambda i: (0, 0)),
        out_shape=jax.ShapeDtypeStruct((128, 128), jnp.float32),
    )(rows_f)


def _dis(deg):
    return jnp.where(deg > 0, lax.rsqrt(jnp.maximum(deg, 1.0)), 0.0)


def _scale_matmul_body(deg_ref, x_ref, w_ref, z_ref):
    dis = _dis(deg_ref[...])
    y = jnp.dot(x_ref[...], w_ref[...], preferred_element_type=jnp.float32)
    z_ref[...] = dis * y


def _scale_matmul(deg, x_pad, W):
    blk = 1024
    return pl.pallas_call(
        _scale_matmul_body,
        grid=(NR // blk,),
        in_specs=[pl.BlockSpec((blk, 1), lambda i: (i, 0)),
                  pl.BlockSpec((blk, D), lambda i: (i, 0)),
                  pl.BlockSpec((D, D), lambda i: (0, 0))],
        out_specs=pl.BlockSpec((blk, D), lambda i: (i, 0)),
        out_shape=jax.ShapeDtypeStruct((NR, D), jnp.float32),
    )(deg, x_pad, W)


def _make_sc_gather():
    @functools.partial(
        pl.kernel,
        out_type=jax.ShapeDtypeStruct((E_PAD, D), jnp.float32),
        mesh=_sc_mesh(),
        scratch_types=[
            pltpu.VMEM((PNL, K), jnp.int32),
            pltpu.VMEM((PNL, K), jnp.int32),
            pltpu.VMEM((K, D), jnp.float32),
            pltpu.VMEM((K, D), jnp.float32),
            pltpu.SemaphoreType.DMA,
            pltpu.SemaphoreType.DMA,
        ],
    )
    def gather(col_hbm, z_hbm, g_hbm, cpa, cpb, buf0, buf1, sem0, sem1):
        c = lax.axis_index("c")
        s = lax.axis_index("s")
        w = s * NC + c
        base = w * EPW

        pltpu.sync_copy(col_hbm.at[w, pl.ds(0, PNL)], cpa)
        pltpu.async_copy(z_hbm.at[cpa.at[0]], buf0, sem0)

        pans = [(cpa,), (cpb,)]
        for p in range(NPANEL):
            (cA,) = pans[p % 2]
            (cB,) = pans[(p + 1) % 2]
            pbase = base + p * PNL * K
            if p < NPANEL - 1:
                pltpu.sync_copy(col_hbm.at[w, pl.ds((p + 1) * PNL, PNL)], cB)

            @pl.loop(0, PNL // 2 - 1, unroll=False)
            def pair(i, cA=cA, pbase=pbase):
                j0 = 2 * i
                pltpu.async_copy(z_hbm.at[cA.at[j0 + 1]], buf1, sem1)
                pltpu.make_async_copy(z_hbm.at[cA.at[j0]], buf0, sem0).wait()
                pltpu.sync_copy(buf0, g_hbm.at[pl.ds(pbase + j0 * K, K)])
                pltpu.async_copy(z_hbm.at[cA.at[j0 + 2]], buf0, sem0)
                pltpu.make_async_copy(z_hbm.at[cA.at[j0 + 1]], buf1, sem1).wait()
                pltpu.sync_copy(buf1, g_hbm.at[pl.ds(pbase + (j0 + 1) * K, K)])

            pltpu.async_copy(z_hbm.at[cA.at[PNL - 1]], buf1, sem1)
            pltpu.make_async_copy(z_hbm.at[cA.at[PNL - 2]], buf0, sem0).wait()
            pltpu.sync_copy(buf0, g_hbm.at[pl.ds(pbase + (PNL - 2) * K, K)])
            if p < NPANEL - 1:
                pltpu.async_copy(z_hbm.at[cB.at[0]], buf0, sem0)
            pltpu.make_async_copy(z_hbm.at[cA.at[PNL - 1]], buf1, sem1).wait()
            pltpu.sync_copy(buf1, g_hbm.at[pl.ds(pbase + (PNL - 1) * K, K)])

    return gather


_sc_gather_kernel = _make_sc_gather()


NACC = 8


def _scatter_body(rows_ref, g_ref, *acc_refs):
    @pl.when(pl.program_id(0) == 0)
    def _():
        for a in acc_refs:
            a[...] = jnp.zeros_like(a)

    def body(j, _):
        j4 = NACC * j
        for q in range(NACC):
            r = rows_ref[0, 0, j4 + q]
            acc_refs[q][pl.ds(r, 1), :] += g_ref[pl.ds(j4 + q, 1), :]
        return 0
    lax.fori_loop(0, CS // NACC, body, 0)


def _tc_scatter(rows2, G):
    nch = E_PAD // CS
    return pl.pallas_call(
        _scatter_body,
        grid=(nch,),
        in_specs=[pl.BlockSpec((1, 1, CS), lambda i: (i, 0, 0),
                               memory_space=pltpu.SMEM),
                  pl.BlockSpec((CS, D), lambda i: (i, 0))],
        out_specs=[pl.BlockSpec((NR, D), lambda i: (0, 0))] * NACC,
        out_shape=[jax.ShapeDtypeStruct((NR, D), jnp.float32)] * NACC,
    )(rows2, G)


def _epilogue_body(deg_ref, b_ref, *refs):
    acc_refs = refs[:-1]
    o_ref = refs[-1]
    dis = _dis(deg_ref[...])
    acc = acc_refs[0][...]
    for a in acc_refs[1:]:
        acc = acc + a[...]
    v = dis * acc + b_ref[...]
    o_ref[...] = jnp.where(v >= 0, v, 0.2 * v)


def _epilogue(deg, accs, b2):
    blk = 1000
    fs = pl.BlockSpec((blk, D), lambda i: (i, 0))
    return pl.pallas_call(
        _epilogue_body,
        grid=(N // blk,),
        in_specs=[pl.BlockSpec((blk, 1), lambda i: (i, 0)),
                  pl.BlockSpec((1, D), lambda i: (0, 0))] + [fs] * NACC,
        out_specs=fs,
        out_shape=jax.ShapeDtypeStruct((N, D), jnp.float32),
    )(deg, b2, *accs)


def kernel(x, edge_index, W, b):
    ei = edge_index.astype(jnp.int32)
    pad = E_PAD - E
    row_p = jnp.concatenate([ei[0], jnp.full((pad,), N, jnp.int32)])
    col_p = jnp.concatenate([ei[1], jnp.full((pad,), N, jnp.int32)])

    rows_f = row_p.astype(jnp.float32).reshape(E_PAD // CE, CE, 1)
    deg_mat = _deg_mat(rows_f)
    deg = deg_mat.reshape(128 * 128, 1)[:NR]

    x_pad = jnp.concatenate(
        [x, jnp.zeros((NR - N, D), jnp.float32)], axis=0)
    z = _scale_matmul(deg, x_pad, W)

    col3 = col_p.reshape(NW, CHUNKS, K)
    G = _sc_gather_kernel(col3, z)

    rows2 = row_p.reshape(E_PAD // CS, 1, CS)
    accs = _tc_scatter(rows2, G)

    return _epilogue(deg[:N], [a[:N] for a in accs], b.reshape(1, D))
